# Initial kernel scaffold; baseline (speedup 1.0000x reference)
import jax
import jax.numpy as jnp
from jax.experimental import pallas as pl


def kernel(x, edge_index, W_rel1, b_rel1, W_root1, W_rel2, b_rel2, W_root2):
    raise NotImplementedError("write your pallas kernel here")


# speedup vs baseline: 4.0031x; 4.0031x over previous
import functools

import jax
import jax.numpy as jnp
from jax import lax
from jax.experimental import pallas as pl
from jax.experimental.pallas import tpu as pltpu
from jax.experimental.pallas import tpu_sc as plsc

NUM_NODES = 10000
NUM_EDGES = 320000
FDIM = 128

NCORES = 2
NSUB = 16
NWORK = NCORES * NSUB

CHUNK = 128
CHUNKS_PER_W = 79
EDGES_PER_W = CHUNKS_PER_W * CHUNK
EDGES_PAD = EDGES_PER_W * NWORK
ROWS_PER_SUB = 640
N_PAD = ROWS_PER_SUB * NSUB


def _sc_scatter_body(x_hbm, src_hbm, dst_hbm, out_hbm,
                     sidx, didx, rows, zbuf, acc, sem):
    c = lax.axis_index("c")
    s = lax.axis_index("s")
    w = s * NCORES + c

    z16 = jnp.zeros((16,), jnp.float32)

    def _zero_row(i, _):
        for j in range(FDIM // 16):
            zbuf[i, pl.ds(j * 16, 16)] = z16
        return 0

    lax.fori_loop(0, CHUNK, _zero_row, 0)

    r0 = s * ROWS_PER_SUB
    for t in range(ROWS_PER_SUB // CHUNK):
        pltpu.sync_copy(zbuf, acc.at[pl.ds(r0 + t * CHUNK, CHUNK)])
    plsc.subcore_barrier()

    e_base = w * EDGES_PER_W

    def _edge_chunk(ci, _):
        off = e_base + ci * CHUNK
        pltpu.sync_copy(src_hbm.at[pl.ds(off, CHUNK)], sidx)
        pltpu.sync_copy(dst_hbm.at[pl.ds(off, CHUNK)], didx)
        pltpu.async_copy(x_hbm.at[sidx], rows, sem).wait()
        pltpu.sync_copy(rows, acc.at[didx], add=True)
        return 0

    lax.fori_loop(0, CHUNKS_PER_W, _edge_chunk, 0)
    plsc.subcore_barrier()

    pltpu.sync_copy(acc.at[pl.ds(r0, ROWS_PER_SUB)],
                    out_hbm.at[c, pl.ds(r0, ROWS_PER_SUB)])


def _sc_scatter(x, src_p, dst_p):
    mesh = plsc.VectorSubcoreMesh(core_axis_name="c", subcore_axis_name="s")
    return pl.kernel(
        _sc_scatter_body,
        out_type=jax.ShapeDtypeStruct((NCORES, N_PAD, FDIM), jnp.float32),
        mesh=mesh,
        scratch_types=[
            pltpu.VMEM((CHUNK,), jnp.int32),
            pltpu.VMEM((CHUNK,), jnp.int32),
            pltpu.VMEM((CHUNK, FDIM), jnp.float32),
            pltpu.VMEM((CHUNK, FDIM), jnp.float32),
            pltpu.VMEM_SHARED((N_PAD, FDIM), jnp.float32),
            pltpu.SemaphoreType.DMA,
        ],
    )(x, src_p, dst_p)


def _mm_body(agg_ref, x_ref, wr_ref, b_ref, wt_ref, o_ref):
    agg = agg_ref[0] + agg_ref[1]
    acc = jnp.dot(agg, wr_ref[...], preferred_element_type=jnp.float32)
    acc = acc + jnp.dot(x_ref[...], wt_ref[...], preferred_element_type=jnp.float32)
    o_ref[...] = jnp.maximum(acc + b_ref[...], 0.0)


def _tc_layer(agg, x, w_rel, b_rel, w_root):
    n = x.shape[0]
    bn = 1000
    return pl.pallas_call(
        _mm_body,
        grid=(n // bn,),
        in_specs=[
            pl.BlockSpec((NCORES, bn, FDIM), lambda i: (0, i, 0)),
            pl.BlockSpec((bn, FDIM), lambda i: (i, 0)),
            pl.BlockSpec((FDIM, FDIM), lambda i: (0, 0)),
            pl.BlockSpec((1, FDIM), lambda i: (0, 0)),
            pl.BlockSpec((FDIM, FDIM), lambda i: (0, 0)),
        ],
        out_specs=pl.BlockSpec((bn, FDIM), lambda i: (i, 0)),
        out_shape=jax.ShapeDtypeStruct((n, FDIM), jnp.float32),
    )(agg, x, w_rel, b_rel.reshape(1, FDIM), w_root)


def kernel(x, edge_index, W_rel1, b_rel1, W_root1, W_rel2, b_rel2, W_root2):
    src = edge_index[0]
    dst = edge_index[1]
    pad = EDGES_PAD - NUM_EDGES
    src_p = jnp.concatenate([src, jnp.zeros((pad,), jnp.int32)])
    dst_p = jnp.concatenate([dst, jnp.full((pad,), NUM_NODES, jnp.int32)])

    agg1 = _sc_scatter(x, src_p, dst_p)
    h1 = _tc_layer(agg1, x, W_rel1, b_rel1, W_root1)
    agg2 = _sc_scatter(h1, src_p, dst_p)
    return _tc_layer(agg2, h1, W_rel2, b_rel2, W_root2)
